# Initial kernel scaffold; baseline (speedup 1.0000x reference)
import jax
import jax.numpy as jnp
from jax.experimental import pallas as pl


def kernel(x, edge_index, batch, W1, b1, W2, b2):
    raise NotImplementedError("write your pallas kernel here")


# speedup vs baseline: 3.3355x; 3.3355x over previous
import functools

import jax
import jax.numpy as jnp
from jax import lax
from jax.experimental import pallas as pl
from jax.experimental.pallas import tpu as pltpu
from jax.experimental.pallas import tpu_sc as plsc

_N = 10000
_E = 320000
_D = 128
_G = 128
_PF = 0.3
_PE = 0.3

_CH = 128
_ROWS = 2560
_EPAD = _ROWS * _CH
_NT = 16
_NSC = 2
_RPT = _ROWS // _NT
_RPTH = _ROWS // (_NSC * _NT)
_NACC = 10240
_TRASH = _N
_ZC = _NACC // _NT // _CH

_SLAB = 40

_BLK = 1000
_NBLK = _N // _BLK


def _zero_rows(rows):
    def _body(i, carry):
        for j in range(8):
            rows[i, pl.ds(j * 16, 16)] = jnp.zeros((16,), jnp.float32)
        return carry
    lax.fori_loop(0, _CH, _body, None)


def _agg_loop(table_hbm, src_hbm, d_hbm, idx_s, idx_d, rows, accum, gsem,
              base, nrows):
    def _outer(t, carry):
        off = base + t * _SLAB
        pltpu.sync_copy(src_hbm.at[pl.ds(off, _SLAB)], idx_s)
        pltpu.sync_copy(d_hbm.at[pl.ds(off, _SLAB)], idx_d)

        def _inner(j, c2):
            pltpu.async_copy(table_hbm.at[idx_s.at[j]], rows, gsem).wait()
            pltpu.sync_copy(rows, accum.at[idx_d.at[j]], add=True)
            return c2
        lax.fori_loop(0, _SLAB, _inner, None)
        return carry
    lax.fori_loop(0, nrows // _SLAB, _outer, None)


def _zero_accum(rows, accum, zbase):
    def _body(k, carry):
        pltpu.sync_copy(rows, accum.at[pl.ds(zbase + k * _CH, _CH)])
        return carry
    lax.fori_loop(0, _ZC, _body, None)


def _writeout(rows, accum, out_hbm, slot, zbase):
    def _body(k, carry):
        off = zbase + k * _CH
        pltpu.sync_copy(accum.at[pl.ds(off, _CH)], rows)
        pltpu.sync_copy(rows, out_hbm.at[slot, pl.ds(off, _CH)])
        return carry
    lax.fori_loop(0, _ZC, _body, None)


def _sc_scratch():
    return [
        pltpu.MemorySpace.VMEM_SHARED((_NACC, _D), jnp.float32),
        pltpu.VMEM((_SLAB, _CH), jnp.int32),
        pltpu.VMEM((_SLAB, _CH), jnp.int32),
        pltpu.VMEM((_CH, _D), jnp.float32),
        pltpu.SemaphoreType.DMA,
    ]


def _sc_mesh():
    return plsc.VectorSubcoreMesh(core_axis_name="c", subcore_axis_name="s")


def _sc_layer1(x, src2d, dstA2d, dstB2d):
    @functools.partial(
        pl.kernel,
        out_type=jax.ShapeDtypeStruct((2, _NACC, _D), jnp.float32),
        mesh=_sc_mesh(),
        scratch_types=_sc_scratch(),
    )
    def _k(x_hbm, src_hbm, dA_hbm, dB_hbm, out_hbm, accum, idx_s, idx_d, rows, gsem):
        c = lax.axis_index("c")
        s = lax.axis_index("s")
        zbase = s * (_NACC // _NT)

        _zero_rows(rows)
        _zero_accum(rows, accum, zbase)
        plsc.subcore_barrier()

        base = s * _RPT

        @pl.when(c == 0)
        def _():
            _agg_loop(x_hbm, src_hbm, dA_hbm, idx_s, idx_d, rows, accum,
                      gsem, base, _RPT)

        @pl.when(c == 1)
        def _():
            _agg_loop(x_hbm, src_hbm, dB_hbm, idx_s, idx_d, rows, accum,
                      gsem, base, _RPT)

        plsc.subcore_barrier()
        _writeout(rows, accum, out_hbm, c, zbase)

    return _k(x, src2d, dstA2d, dstB2d)


def _sc_layer2(h0, h1, h2, src2d, dstA2d, dstB2d):
    @functools.partial(
        pl.kernel,
        out_type=jax.ShapeDtypeStruct((4, _NACC, _D), jnp.float32),
        mesh=_sc_mesh(),
        scratch_types=_sc_scratch(),
    )
    def _k(h0_hbm, h1_hbm, h2_hbm, src_hbm, dA_hbm, dB_hbm, out_hbm,
           accum, idx_s, idx_d, rows, gsem):
        c = lax.axis_index("c")
        s = lax.axis_index("s")
        zbase = s * (_NACC // _NT)

        _zero_rows(rows)
        _zero_accum(rows, accum, zbase)
        plsc.subcore_barrier()

        base = s * _RPT

        @pl.when(c == 0)
        def _():
            _agg_loop(h0_hbm, src_hbm, dA_hbm, idx_s, idx_d, rows, accum,
                      gsem, base, _RPT)

        @pl.when(c == 1)
        def _():
            _agg_loop(h1_hbm, src_hbm, dA_hbm, idx_s, idx_d, rows, accum,
                      gsem, base, _RPT)

        plsc.subcore_barrier()
        _writeout(rows, accum, out_hbm, c, zbase)
        plsc.subcore_barrier()

        _zero_rows(rows)
        _zero_accum(rows, accum, zbase)
        plsc.subcore_barrier()

        baseB = c * (_ROWS // 2) + s * _RPTH
        _agg_loop(h2_hbm, src_hbm, dB_hbm, idx_s, idx_d, rows, accum,
                  gsem, baseB, _RPTH)
        plsc.subcore_barrier()
        _writeout(rows, accum, out_hbm, 2 + c, zbase)

    return _k(h0, h1, h2, src2d, dstA2d, dstB2d)


def _tc_layer1(A, A2, W1, W1m, b1):
    def _body(a_ref, a2_ref, w_ref, wm_ref, b_ref, h0_ref, h1_ref, h2_ref):
        a = a_ref[...]
        a2 = a2_ref[...]
        w = w_ref[...]
        wm = wm_ref[...]
        b = b_ref[...]
        h0_ref[...] = jnp.maximum(
            jnp.dot(a, w, preferred_element_type=jnp.float32) + b, 0.0)
        h1_ref[...] = jnp.maximum(
            jnp.dot(a, wm, preferred_element_type=jnp.float32) + b, 0.0)
        h2_ref[...] = jnp.maximum(
            jnp.dot(a2, w, preferred_element_type=jnp.float32) + b, 0.0)

    row = pl.BlockSpec((_BLK, _D), lambda i: (i, 0))
    full = pl.BlockSpec((_D, _D), lambda i: (0, 0))
    bias = pl.BlockSpec((1, _D), lambda i: (0, 0))
    return pl.pallas_call(
        _body,
        grid=(_NBLK,),
        in_specs=[row, row, full, full, bias],
        out_specs=[row, row, row],
        out_shape=[jax.ShapeDtypeStruct((_N, _D), jnp.float32)] * 3,
    )(A, A2, W1, W1m, b1)


def _tc_pool(Bz, Bz1, Bz2a, Bz2b, W2, b2, batch3d):
    def _body(bz_ref, b1_ref, b2a_ref, b2b_ref, w_ref, bb_ref, bat_ref,
              oz_ref, o1_ref, o2_ref, cnt_ref):
        i = pl.program_id(0)

        @pl.when(i == 0)
        def _():
            oz_ref[...] = jnp.zeros_like(oz_ref)
            o1_ref[...] = jnp.zeros_like(o1_ref)
            o2_ref[...] = jnp.zeros_like(o2_ref)
            cnt_ref[...] = jnp.zeros_like(cnt_ref)

        w = w_ref[...]
        b = bb_ref[...]
        hz = jnp.maximum(
            jnp.dot(bz_ref[...], w, preferred_element_type=jnp.float32) + b, 0.0)
        h1 = jnp.maximum(
            jnp.dot(b1_ref[...], w, preferred_element_type=jnp.float32) + b, 0.0)
        h2 = jnp.maximum(
            jnp.dot(b2a_ref[...] + b2b_ref[...], w,
                    preferred_element_type=jnp.float32) + b, 0.0)
        bat = bat_ref[0]
        oh = (lax.broadcasted_iota(jnp.int32, (_G, _BLK), 0) == bat
              ).astype(jnp.float32)
        oz_ref[...] += jnp.dot(oh, hz, preferred_element_type=jnp.float32)
        o1_ref[...] += jnp.dot(oh, h1, preferred_element_type=jnp.float32)
        o2_ref[...] += jnp.dot(oh, h2, preferred_element_type=jnp.float32)
        cnt_ref[...] += jnp.dot(oh, jnp.ones((_BLK, _D), jnp.float32),
                                preferred_element_type=jnp.float32)

        @pl.when(i == _NBLK - 1)
        def _():
            cnt = jnp.maximum(cnt_ref[...], 1.0)
            oz_ref[...] = oz_ref[...] / cnt
            o1_ref[...] = o1_ref[...] / cnt
            o2_ref[...] = o2_ref[...] / cnt

    row = pl.BlockSpec((_BLK, _D), lambda i: (i, 0))
    full = pl.BlockSpec((_D, _D), lambda i: (0, 0))
    bias = pl.BlockSpec((1, _D), lambda i: (0, 0))
    batb = pl.BlockSpec((1, 1, _BLK), lambda i: (i, 0, 0))
    outb = pl.BlockSpec((_G, _D), lambda i: (0, 0))
    return pl.pallas_call(
        _body,
        grid=(_NBLK,),
        in_specs=[row, row, row, row, full, bias, batb],
        out_specs=[outb, outb, outb],
        out_shape=[jax.ShapeDtypeStruct((_G, _D), jnp.float32)] * 3,
        scratch_shapes=[pltpu.VMEM((_G, _D), jnp.float32)],
    )(Bz, Bz1, Bz2a, Bz2b, W2, b2, batch3d)


def kernel(x, edge_index, batch, W1, b1, W2, b2):
    ka, kb = jax.random.split(jax.random.key(42))
    feat_mask = (jax.random.uniform(ka, (_D,)) > _PF).astype(jnp.float32)
    ew2 = jax.random.uniform(kb, (_E,)) > _PE
    W1m = feat_mask[:, None] * W1

    src = edge_index[0]
    dst = edge_index[1]
    dst2 = jnp.where(ew2, dst, _TRASH)
    pad = _EPAD - _E
    src2d = jnp.concatenate([src, jnp.zeros((pad,), jnp.int32)]).reshape(_ROWS, _CH)
    dA2d = jnp.concatenate([dst, jnp.full((pad,), _TRASH, jnp.int32)]).reshape(_ROWS, _CH)
    dB2d = jnp.concatenate([dst2, jnp.full((pad,), _TRASH, jnp.int32)]).reshape(_ROWS, _CH)

    agg1 = _sc_layer1(x, src2d, dA2d, dB2d)
    h0, h1, h2 = _tc_layer1(agg1[0, :_N], agg1[1, :_N], W1, W1m,
                            b1.reshape(1, _D))
    agg2 = _sc_layer2(h0, h1, h2, src2d, dA2d, dB2d)
    batch3d = batch.reshape(_NBLK, 1, _BLK)
    z, z1, z2 = _tc_pool(agg2[0, :_N], agg2[1, :_N], agg2[2, :_N],
                         agg2[3, :_N], W2, b2.reshape(1, _D), batch3d)
    return (z, z1, z2)


# speedup vs baseline: 3.7878x; 1.1356x over previous
import functools

import jax
import jax.numpy as jnp
from jax import lax
from jax.experimental import pallas as pl
from jax.experimental.pallas import tpu as pltpu
from jax.experimental.pallas import tpu_sc as plsc

_N = 10000
_E = 320000
_D = 128
_G = 128
_PF = 0.3
_PE = 0.3

_CH = 128
_ROWS = 2560
_EPAD = _ROWS * _CH
_NT = 16
_NSC = 2
_RPT = _ROWS // _NT
_RPTH = _ROWS // (_NSC * _NT)
_NACC = 10240
_TRASH = _N
_ZC = _NACC // _NT // _CH

_SLAB = 40

_BLK = 1000
_NBLK = _N // _BLK


def _zero_rows(rows):
    def _body(i, carry):
        for j in range(8):
            rows[i, pl.ds(j * 16, 16)] = jnp.zeros((16,), jnp.float32)
        return carry
    lax.fori_loop(0, _CH, _body, None)


def _agg_loop(table_hbm, src_hbm, d_hbm, idx_s, idx_d, r0, r1, g0, g1,
              accum, base, nrows):
    def _outer(t, carry):
        off = base + t * _SLAB
        pltpu.sync_copy(src_hbm.at[pl.ds(off, _SLAB)], idx_s)
        pltpu.sync_copy(d_hbm.at[pl.ds(off, _SLAB)], idx_d)
        pltpu.async_copy(table_hbm.at[idx_s.at[0]], r0, g0)

        def _inner(u, c2):
            j0 = 2 * u
            j1 = j0 + 1
            j2 = j0 + 2
            pltpu.async_copy(table_hbm.at[idx_s.at[j1]], r1, g1)
            pltpu.make_async_copy(table_hbm.at[idx_s.at[j0]], r0, g0).wait()
            pltpu.sync_copy(r0, accum.at[idx_d.at[j0]], add=True)

            @pl.when(j2 < _SLAB)
            def _():
                pltpu.async_copy(table_hbm.at[idx_s.at[j2]], r0, g0)

            pltpu.make_async_copy(table_hbm.at[idx_s.at[j1]], r1, g1).wait()
            pltpu.sync_copy(r1, accum.at[idx_d.at[j1]], add=True)
            return c2
        lax.fori_loop(0, _SLAB // 2, _inner, None)
        return carry
    lax.fori_loop(0, nrows // _SLAB, _outer, None)


def _zero_accum(rows, accum, zbase):
    def _body(k, carry):
        pltpu.sync_copy(rows, accum.at[pl.ds(zbase + k * _CH, _CH)])
        return carry
    lax.fori_loop(0, _ZC, _body, None)


def _writeout(rows, accum, out_hbm, slot, zbase):
    def _body(k, carry):
        off = zbase + k * _CH
        pltpu.sync_copy(accum.at[pl.ds(off, _CH)], rows)
        pltpu.sync_copy(rows, out_hbm.at[slot, pl.ds(off, _CH)])
        return carry
    lax.fori_loop(0, _ZC, _body, None)


def _sc_scratch():
    return [
        pltpu.MemorySpace.VMEM_SHARED((_NACC, _D), jnp.float32),
        pltpu.VMEM((_SLAB, _CH), jnp.int32),
        pltpu.VMEM((_SLAB, _CH), jnp.int32),
        pltpu.VMEM((_CH, _D), jnp.float32),
        pltpu.VMEM((_CH, _D), jnp.float32),
        pltpu.SemaphoreType.DMA,
        pltpu.SemaphoreType.DMA,
    ]


def _sc_mesh():
    return plsc.VectorSubcoreMesh(core_axis_name="c", subcore_axis_name="s")


def _sc_layer1(x, src2d, dstA2d, dstB2d):
    @functools.partial(
        pl.kernel,
        out_type=jax.ShapeDtypeStruct((2, _NACC, _D), jnp.float32),
        mesh=_sc_mesh(),
        scratch_types=_sc_scratch(),
    )
    def _k(x_hbm, src_hbm, dA_hbm, dB_hbm, out_hbm,
           accum, idx_s, idx_d, r0, r1, g0, g1):
        c = lax.axis_index("c")
        s = lax.axis_index("s")
        zbase = s * (_NACC // _NT)

        _zero_rows(r0)
        _zero_accum(r0, accum, zbase)
        plsc.subcore_barrier()

        base = s * _RPT

        @pl.when(c == 0)
        def _():
            _agg_loop(x_hbm, src_hbm, dA_hbm, idx_s, idx_d, r0, r1, g0, g1,
                      accum, base, _RPT)

        @pl.when(c == 1)
        def _():
            _agg_loop(x_hbm, src_hbm, dB_hbm, idx_s, idx_d, r0, r1, g0, g1,
                      accum, base, _RPT)

        plsc.subcore_barrier()
        _writeout(r0, accum, out_hbm, c, zbase)

    return _k(x, src2d, dstA2d, dstB2d)


def _sc_layer2(h0, h1, h2, src2d, dstA2d, dstB2d):
    @functools.partial(
        pl.kernel,
        out_type=jax.ShapeDtypeStruct((4, _NACC, _D), jnp.float32),
        mesh=_sc_mesh(),
        scratch_types=_sc_scratch(),
    )
    def _k(h0_hbm, h1_hbm, h2_hbm, src_hbm, dA_hbm, dB_hbm, out_hbm,
           accum, idx_s, idx_d, r0, r1, g0, g1):
        c = lax.axis_index("c")
        s = lax.axis_index("s")
        zbase = s * (_NACC // _NT)

        _zero_rows(r0)
        _zero_accum(r0, accum, zbase)
        plsc.subcore_barrier()

        base = s * _RPT

        @pl.when(c == 0)
        def _():
            _agg_loop(h0_hbm, src_hbm, dA_hbm, idx_s, idx_d, r0, r1, g0, g1,
                      accum, base, _RPT)

        @pl.when(c == 1)
        def _():
            _agg_loop(h1_hbm, src_hbm, dA_hbm, idx_s, idx_d, r0, r1, g0, g1,
                      accum, base, _RPT)

        plsc.subcore_barrier()
        _writeout(r0, accum, out_hbm, c, zbase)
        plsc.subcore_barrier()

        _zero_rows(r0)
        _zero_accum(r0, accum, zbase)
        plsc.subcore_barrier()

        baseB = c * (_ROWS // 2) + s * _RPTH
        _agg_loop(h2_hbm, src_hbm, dB_hbm, idx_s, idx_d, r0, r1, g0, g1,
                  accum, baseB, _RPTH)
        plsc.subcore_barrier()
        _writeout(r0, accum, out_hbm, 2 + c, zbase)

    return _k(h0, h1, h2, src2d, dstA2d, dstB2d)


def _tc_layer1(A, A2, W1, W1m, b1):
    def _body(a_ref, a2_ref, w_ref, wm_ref, b_ref, h0_ref, h1_ref, h2_ref):
        a = a_ref[...]
        a2 = a2_ref[...]
        w = w_ref[...]
        wm = wm_ref[...]
        b = b_ref[...]
        h0_ref[...] = jnp.maximum(
            jnp.dot(a, w, preferred_element_type=jnp.float32) + b, 0.0)
        h1_ref[...] = jnp.maximum(
            jnp.dot(a, wm, preferred_element_type=jnp.float32) + b, 0.0)
        h2_ref[...] = jnp.maximum(
            jnp.dot(a2, w, preferred_element_type=jnp.float32) + b, 0.0)

    row = pl.BlockSpec((_BLK, _D), lambda i: (i, 0))
    full = pl.BlockSpec((_D, _D), lambda i: (0, 0))
    bias = pl.BlockSpec((1, _D), lambda i: (0, 0))
    return pl.pallas_call(
        _body,
        grid=(_NBLK,),
        in_specs=[row, row, full, full, bias],
        out_specs=[row, row, row],
        out_shape=[jax.ShapeDtypeStruct((_N, _D), jnp.float32)] * 3,
    )(A, A2, W1, W1m, b1)


def _tc_pool(Bz, Bz1, Bz2a, Bz2b, W2, b2, batch3d):
    def _body(bz_ref, b1_ref, b2a_ref, b2b_ref, w_ref, bb_ref, bat_ref,
              oz_ref, o1_ref, o2_ref, cnt_ref):
        i = pl.program_id(0)

        @pl.when(i == 0)
        def _():
            oz_ref[...] = jnp.zeros_like(oz_ref)
            o1_ref[...] = jnp.zeros_like(o1_ref)
            o2_ref[...] = jnp.zeros_like(o2_ref)
            cnt_ref[...] = jnp.zeros_like(cnt_ref)

        w = w_ref[...]
        b = bb_ref[...]
        hz = jnp.maximum(
            jnp.dot(bz_ref[...], w, preferred_element_type=jnp.float32) + b, 0.0)
        h1 = jnp.maximum(
            jnp.dot(b1_ref[...], w, preferred_element_type=jnp.float32) + b, 0.0)
        h2 = jnp.maximum(
            jnp.dot(b2a_ref[...] + b2b_ref[...], w,
                    preferred_element_type=jnp.float32) + b, 0.0)
        bat = bat_ref[0]
        oh = (lax.broadcasted_iota(jnp.int32, (_G, _BLK), 0) == bat
              ).astype(jnp.float32)
        oz_ref[...] += jnp.dot(oh, hz, preferred_element_type=jnp.float32)
        o1_ref[...] += jnp.dot(oh, h1, preferred_element_type=jnp.float32)
        o2_ref[...] += jnp.dot(oh, h2, preferred_element_type=jnp.float32)
        cnt_ref[...] += jnp.dot(oh, jnp.ones((_BLK, _D), jnp.float32),
                                preferred_element_type=jnp.float32)

        @pl.when(i == _NBLK - 1)
        def _():
            cnt = jnp.maximum(cnt_ref[...], 1.0)
            oz_ref[...] = oz_ref[...] / cnt
            o1_ref[...] = o1_ref[...] / cnt
            o2_ref[...] = o2_ref[...] / cnt

    row = pl.BlockSpec((_BLK, _D), lambda i: (i, 0))
    full = pl.BlockSpec((_D, _D), lambda i: (0, 0))
    bias = pl.BlockSpec((1, _D), lambda i: (0, 0))
    batb = pl.BlockSpec((1, 1, _BLK), lambda i: (i, 0, 0))
    outb = pl.BlockSpec((_G, _D), lambda i: (0, 0))
    return pl.pallas_call(
        _body,
        grid=(_NBLK,),
        in_specs=[row, row, row, row, full, bias, batb],
        out_specs=[outb, outb, outb],
        out_shape=[jax.ShapeDtypeStruct((_G, _D), jnp.float32)] * 3,
        scratch_shapes=[pltpu.VMEM((_G, _D), jnp.float32)],
    )(Bz, Bz1, Bz2a, Bz2b, W2, b2, batch3d)


def kernel(x, edge_index, batch, W1, b1, W2, b2):
    ka, kb = jax.random.split(jax.random.key(42))
    feat_mask = (jax.random.uniform(ka, (_D,)) > _PF).astype(jnp.float32)
    ew2 = jax.random.uniform(kb, (_E,)) > _PE
    W1m = feat_mask[:, None] * W1

    src = edge_index[0]
    dst = edge_index[1]
    dst2 = jnp.where(ew2, dst, _TRASH)
    pad = _EPAD - _E
    src2d = jnp.concatenate([src, jnp.zeros((pad,), jnp.int32)]).reshape(_ROWS, _CH)
    dA2d = jnp.concatenate([dst, jnp.full((pad,), _TRASH, jnp.int32)]).reshape(_ROWS, _CH)
    dB2d = jnp.concatenate([dst2, jnp.full((pad,), _TRASH, jnp.int32)]).reshape(_ROWS, _CH)

    agg1 = _sc_layer1(x, src2d, dA2d, dB2d)
    h0, h1, h2 = _tc_layer1(agg1[0, :_N], agg1[1, :_N], W1, W1m,
                            b1.reshape(1, _D))
    agg2 = _sc_layer2(h0, h1, h2, src2d, dA2d, dB2d)
    batch3d = batch.reshape(_NBLK, 1, _BLK)
    z, z1, z2 = _tc_pool(agg2[0, :_N], agg2[1, :_N], agg2[2, :_N],
                         agg2[3, :_N], W2, b2.reshape(1, _D), batch3d)
    return (z, z1, z2)
